# Initial kernel scaffold; baseline (speedup 1.0000x reference)
import jax
import jax.numpy as jnp
from jax.experimental import pallas as pl
from jax.experimental.pallas import tpu as pltpu

INPUT_SIZE = 1024
OUTPUT_SIZE = 1024
NUM_CELLS = 16
TOP_K = 2
BATCH = 32
FEAT = INPUT_SIZE + OUTPUT_SIZE
BLOCK_N = 2 * OUTPUT_SIZE
N_STEPS = NUM_CELLS * 4 * OUTPUT_SIZE // BLOCK_N


def _body(feats_ref, w_ref, bg_ref, wt_ref, bt_ref, c_ref,
          outc_ref, outh_ref, gate_scr, ij_scr):
    s = pl.program_id(0)

    @pl.when(s == 0)
    def _init():
        logits = jnp.dot(feats_ref[:, :], wt_ref[:, :],
                         preferred_element_type=jnp.float32)
        logits = logits + bt_ref[0, :, :]
        idx1 = jnp.argmax(logits, axis=-1)[:, None]
        cols = jax.lax.broadcasted_iota(jnp.int32, (BATCH, NUM_CELLS), 1)
        oh1 = (cols == idx1)
        m1 = jnp.max(logits, axis=-1, keepdims=True)
        masked = jnp.where(oh1, -jnp.inf, logits)
        idx2 = jnp.argmax(masked, axis=-1)[:, None]
        oh2 = (cols == idx2)
        m2 = jnp.max(masked, axis=-1, keepdims=True)
        e2 = jnp.exp(m2 - m1)
        p1 = 1.0 / (1.0 + e2)
        p2 = e2 / (1.0 + e2)
        gate_scr[:, :] = jnp.where(oh1, p1, 0.0) + jnp.where(oh2, p2, 0.0)
        outc_ref[:, :] = jnp.zeros_like(outc_ref)
        outh_ref[:, :] = jnp.zeros_like(outh_ref)

    block = jnp.dot(feats_ref[:, :], w_ref[:, :],
                    preferred_element_type=jnp.float32)
    block = block + bg_ref[0, :, :]

    @pl.when(s % 2 == 0)
    def _ij():
        ij_scr[:, :] = (jax.nn.sigmoid(block[:, :OUTPUT_SIZE])
                        * jnp.tanh(block[:, OUTPUT_SIZE:]))

    @pl.when(s % 2 == 1)
    def _fo():
        f = jax.nn.sigmoid(block[:, :OUTPUT_SIZE])
        o = jax.nn.sigmoid(block[:, OUTPUT_SIZE:])
        new_c = f * c_ref[:, :] + ij_scr[:, :]
        new_h = o * jnp.tanh(new_c)
        cell = s // 2
        rows = jax.lax.broadcasted_iota(jnp.int32, (NUM_CELLS, 1), 0)
        onehot = (rows == cell).astype(jnp.float32)
        g = jnp.dot(gate_scr[:, :], onehot,
                    preferred_element_type=jnp.float32)
        outc_ref[:, :] += g * new_c
        outh_ref[:, :] += g * new_h


@jax.jit
def kernel(x, c, h, W_gates, b_gates, W_topk, b_topk):
    feats = jnp.concatenate([x, h], axis=-1)
    bg = b_gates.reshape(N_STEPS, 1, BLOCK_N)
    bt = b_topk.reshape(1, 1, NUM_CELLS)

    grid_spec = pl.GridSpec(
        grid=(N_STEPS,),
        in_specs=[
            pl.BlockSpec((BATCH, FEAT), lambda s: (0, 0)),
            pl.BlockSpec((FEAT, BLOCK_N), lambda s: (0, s)),
            pl.BlockSpec((1, 1, BLOCK_N), lambda s: (s, 0, 0)),
            pl.BlockSpec((FEAT, NUM_CELLS), lambda s: (0, 0)),
            pl.BlockSpec((1, 1, NUM_CELLS), lambda s: (0, 0, 0)),
            pl.BlockSpec((BATCH, OUTPUT_SIZE), lambda s: (0, 0)),
        ],
        out_specs=[
            pl.BlockSpec((BATCH, OUTPUT_SIZE), lambda s: (0, 0)),
            pl.BlockSpec((BATCH, OUTPUT_SIZE), lambda s: (0, 0)),
        ],
    )

    out_c, out_h = pl.pallas_call(
        _body,
        grid_spec=grid_spec,
        out_shape=[
            jax.ShapeDtypeStruct((BATCH, OUTPUT_SIZE), jnp.float32),
            jax.ShapeDtypeStruct((BATCH, OUTPUT_SIZE), jnp.float32),
        ],
        scratch_shapes=[
            pltpu.VMEM((BATCH, NUM_CELLS), jnp.float32),
            pltpu.VMEM((BATCH, OUTPUT_SIZE), jnp.float32),
        ],
        compiler_params=pltpu.CompilerParams(
            dimension_semantics=("arbitrary",),
        ),
    )(feats, W_gates, bg, W_topk, bt, c)

    return (out_h, (out_c, out_h))


# speedup vs baseline: 1.2577x; 1.2577x over previous
import jax
import jax.numpy as jnp
from jax.experimental import pallas as pl
from jax.experimental.pallas import tpu as pltpu

INPUT_SIZE = 1024
OUTPUT_SIZE = 1024
NUM_CELLS = 16
TOP_K = 2
BATCH = 32
FEAT = INPUT_SIZE + OUTPUT_SIZE
BLOCK_N = 2 * OUTPUT_SIZE
N_STEPS = NUM_CELLS * 4 * OUTPUT_SIZE // BLOCK_N


def _body(feats_ref, w_ref, bg_ref, wt_ref, bt_ref, c_ref,
          outc_ref, outh_ref, gate_scr, ij_scr):
    s = pl.program_id(0)

    @pl.when(s == 0)
    def _init():
        logits = jnp.dot(feats_ref[:, :], wt_ref[:, :],
                         preferred_element_type=jnp.float32)
        logits = logits + bt_ref[0, :, :]
        idx1 = jnp.argmax(logits, axis=-1)[:, None]
        cols = jax.lax.broadcasted_iota(jnp.int32, (BATCH, NUM_CELLS), 1)
        oh1 = (cols == idx1)
        m1 = jnp.max(logits, axis=-1, keepdims=True)
        masked = jnp.where(oh1, -jnp.inf, logits)
        idx2 = jnp.argmax(masked, axis=-1)[:, None]
        oh2 = (cols == idx2)
        m2 = jnp.max(masked, axis=-1, keepdims=True)
        e2 = jnp.exp(m2 - m1)
        p1 = 1.0 / (1.0 + e2)
        p2 = e2 / (1.0 + e2)
        gate_scr[:, :] = jnp.where(oh1, p1, 0.0) + jnp.where(oh2, p2, 0.0)
        outc_ref[:, :] = jnp.zeros_like(outc_ref)
        outh_ref[:, :] = jnp.zeros_like(outh_ref)

    block = jnp.dot(feats_ref[:, :], w_ref[:, :],
                    preferred_element_type=jnp.float32)
    block = block + bg_ref[0, :, :]

    @pl.when(s % 2 == 0)
    def _ij():
        ij_scr[:, :] = (jax.nn.sigmoid(block[:, :OUTPUT_SIZE])
                        * jnp.tanh(block[:, OUTPUT_SIZE:]))

    @pl.when(s % 2 == 1)
    def _fo():
        f = jax.nn.sigmoid(block[:, :OUTPUT_SIZE])
        o = jax.nn.sigmoid(block[:, OUTPUT_SIZE:])
        new_c = f * c_ref[:, :] + ij_scr[:, :]
        new_h = o * jnp.tanh(new_c)
        cell = s // 2
        rows = jax.lax.broadcasted_iota(jnp.int32, (NUM_CELLS, 1), 0)
        onehot = (rows == cell).astype(jnp.float32)
        g = jnp.dot(gate_scr[:, :], onehot,
                    preferred_element_type=jnp.float32)
        outc_ref[:, :] += g * new_c
        outh_ref[:, :] += g * new_h


@jax.jit
def kernel(x, c, h, W_gates, b_gates, W_topk, b_topk):
    feats = jnp.concatenate([x, h], axis=-1)
    bg = b_gates.reshape(N_STEPS, 1, BLOCK_N)
    bt = b_topk.reshape(1, 1, NUM_CELLS)

    out_c, out_h = pl.pallas_call(
        _body,
        grid=(N_STEPS,),
        in_specs=[
            pl.BlockSpec((BATCH, FEAT), lambda s: (0, 0)),
            pl.BlockSpec((FEAT, BLOCK_N), lambda s: (0, s)),
            pl.BlockSpec((1, 1, BLOCK_N), lambda s: (s, 0, 0)),
            pl.BlockSpec((FEAT, NUM_CELLS), lambda s: (0, 0)),
            pl.BlockSpec((1, 1, NUM_CELLS), lambda s: (0, 0, 0)),
            pl.BlockSpec((BATCH, OUTPUT_SIZE), lambda s: (0, 0)),
        ],
        out_specs=[
            pl.BlockSpec((BATCH, OUTPUT_SIZE), lambda s: (0, 0)),
            pl.BlockSpec((BATCH, OUTPUT_SIZE), lambda s: (0, 0)),
        ],
        out_shape=[
            jax.ShapeDtypeStruct((BATCH, OUTPUT_SIZE), jnp.float32),
            jax.ShapeDtypeStruct((BATCH, OUTPUT_SIZE), jnp.float32),
        ],
        scratch_shapes=[
            pltpu.VMEM((BATCH, NUM_CELLS), jnp.float32),
            pltpu.VMEM((BATCH, OUTPUT_SIZE), jnp.float32),
        ],
        compiler_params=pltpu.CompilerParams(
            dimension_semantics=("arbitrary",),
        ),
    )(feats, W_gates, bg, W_topk, bt, c)

    return (out_h, (out_c, out_h))


# speedup vs baseline: 1.2695x; 1.0094x over previous
import jax
import jax.numpy as jnp
from jax.experimental import pallas as pl
from jax.experimental.pallas import tpu as pltpu

INPUT_SIZE = 1024
OUTPUT_SIZE = 1024
NUM_CELLS = 16
TOP_K = 2
BATCH = 32
FEAT = INPUT_SIZE + OUTPUT_SIZE
N_STEPS = 2 * NUM_CELLS


def _body(feats_ref, wa_ref, wb_ref, bg_ref, wt_ref, bt_ref, c_ref,
          outc_ref, outh_ref, gate_scr, ij_scr):
    s = pl.program_id(0)

    @pl.when(s == 0)
    def _init():
        logits = jnp.dot(feats_ref[:, :], wt_ref[:, :],
                         preferred_element_type=jnp.float32)
        logits = logits + bt_ref[0, :, :]
        idx1 = jnp.argmax(logits, axis=-1)[:, None]
        cols = jax.lax.broadcasted_iota(jnp.int32, (BATCH, NUM_CELLS), 1)
        oh1 = (cols == idx1)
        m1 = jnp.max(logits, axis=-1, keepdims=True)
        masked = jnp.where(oh1, -jnp.inf, logits)
        idx2 = jnp.argmax(masked, axis=-1)[:, None]
        oh2 = (cols == idx2)
        m2 = jnp.max(masked, axis=-1, keepdims=True)
        e2 = jnp.exp(m2 - m1)
        p1 = 1.0 / (1.0 + e2)
        p2 = e2 / (1.0 + e2)
        gate_scr[:, :] = jnp.where(oh1, p1, 0.0) + jnp.where(oh2, p2, 0.0)
        outc_ref[:, :] = jnp.zeros_like(outc_ref)
        outh_ref[:, :] = jnp.zeros_like(outh_ref)

    feats = feats_ref[:, :]
    ga = jnp.dot(feats, wa_ref[:, :], preferred_element_type=jnp.float32)
    gb = jnp.dot(feats, wb_ref[:, :], preferred_element_type=jnp.float32)
    bg = bg_ref[0, :, :]
    ga = ga + bg[:, :OUTPUT_SIZE]
    gb = gb + bg[:, OUTPUT_SIZE:]

    @pl.when(s % 2 == 0)
    def _ij():
        ij_scr[:, :] = jax.nn.sigmoid(ga) * jnp.tanh(gb)

    @pl.when(s % 2 == 1)
    def _fo():
        new_c = jax.nn.sigmoid(ga) * c_ref[:, :] + ij_scr[:, :]
        new_h = jax.nn.sigmoid(gb) * jnp.tanh(new_c)
        cell = s // 2
        rows = jax.lax.broadcasted_iota(jnp.int32, (NUM_CELLS, 1), 0)
        onehot = (rows == cell).astype(jnp.float32)
        g = jnp.dot(gate_scr[:, :], onehot,
                    preferred_element_type=jnp.float32)
        outc_ref[:, :] += g * new_c
        outh_ref[:, :] += g * new_h


@jax.jit
def kernel(x, c, h, W_gates, b_gates, W_topk, b_topk):
    feats = jnp.concatenate([x, h], axis=-1)
    bg = b_gates.reshape(N_STEPS, 1, 2 * OUTPUT_SIZE)
    bt = b_topk.reshape(1, 1, NUM_CELLS)

    out_c, out_h = pl.pallas_call(
        _body,
        grid=(N_STEPS,),
        in_specs=[
            pl.BlockSpec((BATCH, FEAT), lambda s: (0, 0)),
            pl.BlockSpec((FEAT, OUTPUT_SIZE), lambda s: (0, 2 * s)),
            pl.BlockSpec((FEAT, OUTPUT_SIZE), lambda s: (0, 2 * s + 1)),
            pl.BlockSpec((1, 1, 2 * OUTPUT_SIZE), lambda s: (s, 0, 0)),
            pl.BlockSpec((FEAT, NUM_CELLS), lambda s: (0, 0)),
            pl.BlockSpec((1, 1, NUM_CELLS), lambda s: (0, 0, 0)),
            pl.BlockSpec((BATCH, OUTPUT_SIZE), lambda s: (0, 0)),
        ],
        out_specs=[
            pl.BlockSpec((BATCH, OUTPUT_SIZE), lambda s: (0, 0)),
            pl.BlockSpec((BATCH, OUTPUT_SIZE), lambda s: (0, 0)),
        ],
        out_shape=[
            jax.ShapeDtypeStruct((BATCH, OUTPUT_SIZE), jnp.float32),
            jax.ShapeDtypeStruct((BATCH, OUTPUT_SIZE), jnp.float32),
        ],
        scratch_shapes=[
            pltpu.VMEM((BATCH, NUM_CELLS), jnp.float32),
            pltpu.VMEM((BATCH, OUTPUT_SIZE), jnp.float32),
        ],
        compiler_params=pltpu.CompilerParams(
            dimension_semantics=("arbitrary",),
            vmem_limit_bytes=60 * 1024 * 1024,
        ),
    )(feats, W_gates, W_gates, bg, W_topk, bt, c)

    return (out_h, (out_c, out_h))


# speedup vs baseline: 1.3181x; 1.0383x over previous
import jax
import jax.numpy as jnp
from jax.experimental import pallas as pl
from jax.experimental.pallas import tpu as pltpu

INPUT_SIZE = 1024
OUTPUT_SIZE = 1024
NUM_CELLS = 16
TOP_K = 2
BATCH = 32
FEAT = INPUT_SIZE + OUTPUT_SIZE
HALF = OUTPUT_SIZE // 2


def _body(feats_ref, wi_ref, wj_ref, wf_ref, wo_ref, bg_ref, wt_ref, bt_ref,
          c_ref, outc_ref, outh_ref, gate_scr):
    half = pl.program_id(0)
    e = pl.program_id(1)

    @pl.when(e == 0)
    def _init():
        logits = jnp.dot(feats_ref[:, :], wt_ref[:, :],
                         preferred_element_type=jnp.float32)
        logits = logits + bt_ref[0, :, :]
        idx1 = jnp.argmax(logits, axis=-1)[:, None]
        cols = jax.lax.broadcasted_iota(jnp.int32, (BATCH, NUM_CELLS), 1)
        oh1 = (cols == idx1)
        m1 = jnp.max(logits, axis=-1, keepdims=True)
        masked = jnp.where(oh1, -jnp.inf, logits)
        idx2 = jnp.argmax(masked, axis=-1)[:, None]
        oh2 = (cols == idx2)
        m2 = jnp.max(masked, axis=-1, keepdims=True)
        e2 = jnp.exp(m2 - m1)
        p1 = 1.0 / (1.0 + e2)
        p2 = e2 / (1.0 + e2)
        gate_scr[:, :] = jnp.where(oh1, p1, 0.0) + jnp.where(oh2, p2, 0.0)
        outc_ref[:, :] = jnp.zeros_like(outc_ref)
        outh_ref[:, :] = jnp.zeros_like(outh_ref)

    feats = feats_ref[:, :]
    gi = jnp.dot(feats, wi_ref[:, :], preferred_element_type=jnp.float32)
    gj = jnp.dot(feats, wj_ref[:, :], preferred_element_type=jnp.float32)
    gf = jnp.dot(feats, wf_ref[:, :], preferred_element_type=jnp.float32)
    go = jnp.dot(feats, wo_ref[:, :], preferred_element_type=jnp.float32)

    def bias(g):
        return jnp.where(half == 0,
                         bg_ref[0, 2 * g:2 * g + 1, :],
                         bg_ref[0, 2 * g + 1:2 * g + 2, :])
    gi = gi + bias(0)
    gj = gj + bias(1)
    gf = gf + bias(2)
    go = go + bias(3)

    new_c = jax.nn.sigmoid(gf) * c_ref[:, :] + jax.nn.sigmoid(gi) * jnp.tanh(gj)
    new_h = jax.nn.sigmoid(go) * jnp.tanh(new_c)

    rows = jax.lax.broadcasted_iota(jnp.int32, (NUM_CELLS, 1), 0)
    onehot = (rows == e).astype(jnp.float32)
    g = jnp.dot(gate_scr[:, :], onehot,
                preferred_element_type=jnp.float32)
    outc_ref[:, :] += g * new_c
    outh_ref[:, :] += g * new_h


@jax.jit
def kernel(x, c, h, W_gates, b_gates, W_topk, b_topk):
    feats = jnp.concatenate([x, h], axis=-1)
    bg = b_gates.reshape(NUM_CELLS, 8, HALF)
    bt = b_topk.reshape(1, 1, NUM_CELLS)

    wspec = lambda g: pl.BlockSpec(
        (FEAT, HALF), lambda m, e, g=g: (0, 2 * (4 * e + g) + m))

    out_c, out_h = pl.pallas_call(
        _body,
        grid=(2, NUM_CELLS),
        in_specs=[
            pl.BlockSpec((BATCH, FEAT), lambda m, e: (0, 0)),
            wspec(0), wspec(1), wspec(2), wspec(3),
            pl.BlockSpec((1, 8, HALF), lambda m, e: (e, 0, 0)),
            pl.BlockSpec((FEAT, NUM_CELLS), lambda m, e: (0, 0)),
            pl.BlockSpec((1, 1, NUM_CELLS), lambda m, e: (0, 0, 0)),
            pl.BlockSpec((BATCH, HALF), lambda m, e: (0, m)),
        ],
        out_specs=[
            pl.BlockSpec((BATCH, HALF), lambda m, e: (0, m)),
            pl.BlockSpec((BATCH, HALF), lambda m, e: (0, m)),
        ],
        out_shape=[
            jax.ShapeDtypeStruct((BATCH, OUTPUT_SIZE), jnp.float32),
            jax.ShapeDtypeStruct((BATCH, OUTPUT_SIZE), jnp.float32),
        ],
        scratch_shapes=[
            pltpu.VMEM((BATCH, NUM_CELLS), jnp.float32),
        ],
        compiler_params=pltpu.CompilerParams(
            dimension_semantics=("arbitrary", "arbitrary"),
            vmem_limit_bytes=60 * 1024 * 1024,
        ),
    )(feats, W_gates, W_gates, W_gates, W_gates, bg, W_topk, bt, c)

    return (out_h, (out_c, out_h))


# speedup vs baseline: 1.3203x; 1.0017x over previous
import jax
import jax.numpy as jnp
from jax.experimental import pallas as pl
from jax.experimental.pallas import tpu as pltpu

INPUT_SIZE = 1024
OUTPUT_SIZE = 1024
NUM_CELLS = 16
TOP_K = 2
BATCH = 32
FEAT = INPUT_SIZE + OUTPUT_SIZE
HALF = OUTPUT_SIZE // 2


def _body(feats_ref, wi0_ref, wi1_ref, wj0_ref, wj1_ref, wf0_ref, wf1_ref,
          wo0_ref, wo1_ref, bg_ref, wt_ref, bt_ref,
          c_ref, outc_ref, outh_ref, gate_scr):
    half = pl.program_id(0)
    e = pl.program_id(1)

    @pl.when(e == 0)
    def _init():
        logits = jnp.dot(feats_ref[:, :], wt_ref[:, :],
                         preferred_element_type=jnp.float32)
        logits = logits + bt_ref[0, :, :]
        idx1 = jnp.argmax(logits, axis=-1)[:, None]
        cols = jax.lax.broadcasted_iota(jnp.int32, (BATCH, NUM_CELLS), 1)
        oh1 = (cols == idx1)
        m1 = jnp.max(logits, axis=-1, keepdims=True)
        masked = jnp.where(oh1, -jnp.inf, logits)
        idx2 = jnp.argmax(masked, axis=-1)[:, None]
        oh2 = (cols == idx2)
        m2 = jnp.max(masked, axis=-1, keepdims=True)
        e2 = jnp.exp(m2 - m1)
        p1 = 1.0 / (1.0 + e2)
        p2 = e2 / (1.0 + e2)
        gate_scr[:, :] = jnp.where(oh1, p1, 0.0) + jnp.where(oh2, p2, 0.0)
        outc_ref[:, :] = jnp.zeros_like(outc_ref)
        outh_ref[:, :] = jnp.zeros_like(outh_ref)

    feats = feats_ref[:, :]

    def mm(a_ref, b_ref):
        return jnp.concatenate(
            [jnp.dot(feats, a_ref[:, :], preferred_element_type=jnp.float32),
             jnp.dot(feats, b_ref[:, :], preferred_element_type=jnp.float32)],
            axis=1)

    gi = mm(wi0_ref, wi1_ref)
    gj = mm(wj0_ref, wj1_ref)
    gf = mm(wf0_ref, wf1_ref)
    go = mm(wo0_ref, wo1_ref)

    def bias(g):
        return jnp.where(half == 0,
                         bg_ref[0, 2 * g:2 * g + 1, :],
                         bg_ref[0, 2 * g + 1:2 * g + 2, :])
    gi = gi + bias(0)
    gj = gj + bias(1)
    gf = gf + bias(2)
    go = go + bias(3)

    new_c = jax.nn.sigmoid(gf) * c_ref[:, :] + jax.nn.sigmoid(gi) * jnp.tanh(gj)
    new_h = jax.nn.sigmoid(go) * jnp.tanh(new_c)

    rows = jax.lax.broadcasted_iota(jnp.int32, (NUM_CELLS, 1), 0)
    onehot = (rows == e).astype(jnp.float32)
    g = jnp.dot(gate_scr[:, :], onehot,
                preferred_element_type=jnp.float32)
    outc_ref[:, :] += g * new_c
    outh_ref[:, :] += g * new_h


@jax.jit
def kernel(x, c, h, W_gates, b_gates, W_topk, b_topk):
    feats = jnp.concatenate([x, h], axis=-1)
    bg = b_gates.reshape(NUM_CELLS, 8, HALF)
    bt = b_topk.reshape(1, 1, NUM_CELLS)

    Q = HALF // 2
    wspec = lambda g, lr: pl.BlockSpec(
        (FEAT, Q), lambda m, e, g=g, lr=lr: (0, 4 * (4 * e + g) + 2 * m + lr))

    out_c, out_h = pl.pallas_call(
        _body,
        grid=(2, NUM_CELLS),
        in_specs=[
            pl.BlockSpec((BATCH, FEAT), lambda m, e: (0, 0)),
            wspec(0, 0), wspec(0, 1), wspec(1, 0), wspec(1, 1),
            wspec(2, 0), wspec(2, 1), wspec(3, 0), wspec(3, 1),
            pl.BlockSpec((1, 8, HALF), lambda m, e: (e, 0, 0)),
            pl.BlockSpec((FEAT, NUM_CELLS), lambda m, e: (0, 0)),
            pl.BlockSpec((1, 1, NUM_CELLS), lambda m, e: (0, 0, 0)),
            pl.BlockSpec((BATCH, HALF), lambda m, e: (0, m)),
        ],
        out_specs=[
            pl.BlockSpec((BATCH, HALF), lambda m, e: (0, m)),
            pl.BlockSpec((BATCH, HALF), lambda m, e: (0, m)),
        ],
        out_shape=[
            jax.ShapeDtypeStruct((BATCH, OUTPUT_SIZE), jnp.float32),
            jax.ShapeDtypeStruct((BATCH, OUTPUT_SIZE), jnp.float32),
        ],
        scratch_shapes=[
            pltpu.VMEM((BATCH, NUM_CELLS), jnp.float32),
        ],
        compiler_params=pltpu.CompilerParams(
            dimension_semantics=("arbitrary", "arbitrary"),
            vmem_limit_bytes=60 * 1024 * 1024,
        ),
    )(feats, W_gates, W_gates, W_gates, W_gates,
      W_gates, W_gates, W_gates, W_gates, bg, W_topk, bt, c)

    return (out_h, (out_c, out_h))
